# baseline (speedup 1.0000x reference)
import functools

import jax
import jax.numpy as jnp
from jax import lax
from jax.experimental import pallas as pl
from jax.experimental.pallas import tpu as pltpu
from jax.experimental.pallas import tpu_sc as plsc

_B = 4096
_D = 64
_K = 512
_T = 16


def _tc_body(x_ref, idx_ref, l_ref, out_ref):
    t = pl.program_id(0)
    x = x_ref[...]
    idxv = idx_ref[...]
    lt = l_ref[0]
    n2 = jnp.sum(lt * lt, axis=1, keepdims=True)
    ln = lt * lax.rsqrt(jnp.maximum(n2, 1e-12))
    _RC = 16
    _RB = _B // _RC
    parts = []
    for r in range(_RC):
        s = lax.dot_general(
            x[r * _RB:(r + 1) * _RB], ln, (((1,), (1,)), ((), ())),
            preferred_element_type=jnp.float32,
            precision=lax.Precision.HIGHEST,
        )
        parts.append(jnp.argmax(s, axis=1).astype(jnp.int32))
    a = jnp.concatenate(parts)[:, None]

    @pl.when(t == 0)
    def _():
        out_ref[...] = jnp.zeros_like(out_ref)

    out_ref[...] = jnp.where(idxv == t, t * _K + a, out_ref[...])


def _best_ids(x, idx2, latent_vectors):
    return pl.pallas_call(
        _tc_body,
        grid=(_T,),
        in_specs=[
            pl.BlockSpec((_B, _D), lambda t: (0, 0)),
            pl.BlockSpec((_B, 1), lambda t: (0, 0)),
            pl.BlockSpec((1, _K, _D), lambda t: (t, 0, 0)),
        ],
        out_specs=pl.BlockSpec((_B, 1), lambda t: (0, 0)),
        out_shape=jax.ShapeDtypeStruct((_B, 1), jnp.int32),
    )(x, idx2, latent_vectors)


_NW = 32
_BPW = _B // _NW


def _sc_gather(table, ids):
    mesh = plsc.VectorSubcoreMesh(core_axis_name="c", subcore_axis_name="s")

    @functools.partial(
        pl.kernel,
        mesh=mesh,
        compiler_params=pltpu.CompilerParams(use_tc_tiling_on_sc=False),
        out_type=jax.ShapeDtypeStruct((_B, _D), jnp.float32),
        scratch_types=[
            pltpu.VMEM((_BPW,), jnp.int32),
            pltpu.VMEM((_BPW, _D), jnp.float32),
            pltpu.SemaphoreType.DMA,
        ],
    )
    def k(table_hbm, idx_hbm, out_hbm, idx_v, rows_v, sem):
        wid = lax.axis_index("s") * 2 + lax.axis_index("c")
        base = wid * _BPW
        pltpu.sync_copy(idx_hbm.at[pl.ds(base, _BPW)], idx_v)
        pltpu.async_copy(table_hbm.at[idx_v], rows_v, sem).wait()
        pltpu.sync_copy(rows_v, out_hbm.at[pl.ds(base, _BPW)])

    return k(table, ids)


def kernel(inputs, index, latent_vectors):
    x = inputs[:, :, 0]
    idx2 = index[:, None].astype(jnp.int32)
    best = _best_ids(x, idx2, latent_vectors)
    table = latent_vectors.reshape(_T * _K, _D)
    return _sc_gather(table, best[:, 0])


# speedup vs baseline: 1.0094x; 1.0094x over previous
import functools

import jax
import jax.numpy as jnp
from jax import lax
from jax.experimental import pallas as pl
from jax.experimental.pallas import tpu as pltpu
from jax.experimental.pallas import tpu_sc as plsc

_B = 4096
_D = 64
_K = 512
_T = 16


def _tc_body(x_ref, idx_ref, l_ref, out_ref, tpad_ref):
    t = pl.program_id(0)
    x = x_ref[...]
    idxv = idx_ref[...]
    lt = l_ref[0]
    tpad_ref[...] = jnp.concatenate(
        [lt, jnp.zeros((_K, 128 - _D), jnp.float32)], axis=1)
    n2 = jnp.sum(lt * lt, axis=1, keepdims=True)
    ln = lt * lax.rsqrt(jnp.maximum(n2, 1e-12))
    _RC = 16
    _RB = _B // _RC
    parts = []
    for r in range(_RC):
        s = lax.dot_general(
            x[r * _RB:(r + 1) * _RB], ln, (((1,), (1,)), ((), ())),
            preferred_element_type=jnp.float32,
            precision=lax.Precision.HIGHEST,
        )
        parts.append(jnp.argmax(s, axis=1).astype(jnp.int32))
    a = jnp.concatenate(parts)[:, None]

    @pl.when(t == 0)
    def _():
        out_ref[...] = jnp.zeros_like(out_ref)

    out_ref[...] = jnp.where(idxv == t, t * _K + a, out_ref[...])


def _best_ids(x, idx2, latent_vectors):
    return pl.pallas_call(
        _tc_body,
        grid=(_T,),
        in_specs=[
            pl.BlockSpec((_B, _D), lambda t: (0, 0)),
            pl.BlockSpec((_B, 1), lambda t: (0, 0)),
            pl.BlockSpec((1, _K, _D), lambda t: (t, 0, 0)),
        ],
        out_specs=[
            pl.BlockSpec((_B, 1), lambda t: (0, 0)),
            pl.BlockSpec((_K, 128), lambda t: (t, 0)),
        ],
        out_shape=[
            jax.ShapeDtypeStruct((_B, 1), jnp.int32),
            jax.ShapeDtypeStruct((_T * _K, 128), jnp.float32),
        ],
    )(x, idx2, latent_vectors)


_NW = 32
_BPW = _B // _NW


def _sc_gather(table, ids):
    mesh = plsc.VectorSubcoreMesh(core_axis_name="c", subcore_axis_name="s")

    @functools.partial(
        pl.kernel,
        mesh=mesh,
        out_type=jax.ShapeDtypeStruct((_B, 128), jnp.float32),
        scratch_types=[
            pltpu.VMEM((_BPW,), jnp.int32),
            pltpu.VMEM((_BPW, 128), jnp.float32),
            pltpu.SemaphoreType.DMA,
        ],
    )
    def k(table_hbm, idx_hbm, out_hbm, idx_v, rows_v, sem):
        wid = lax.axis_index("s") * 2 + lax.axis_index("c")
        base = wid * _BPW
        pltpu.sync_copy(idx_hbm.at[pl.ds(base, _BPW)], idx_v)
        pltpu.async_copy(table_hbm.at[idx_v], rows_v, sem).wait()
        pltpu.sync_copy(rows_v, out_hbm.at[pl.ds(base, _BPW)])

    return k(table, ids)


def kernel(inputs, index, latent_vectors):
    x = inputs[:, :, 0]
    idx2 = index[:, None].astype(jnp.int32)
    best, tpad = _best_ids(x, idx2, latent_vectors)
    sel = _sc_gather(tpad, best[:, 0])
    return sel[:, :_D]
